# Initial kernel scaffold; baseline (speedup 1.0000x reference)
import jax
import jax.numpy as jnp
from jax.experimental import pallas as pl


def kernel(x, gamma, beta, Wg, W1, b1, W2, b2):
    raise NotImplementedError("write your pallas kernel here")


# speedup vs baseline: 3.0105x; 3.0105x over previous
import functools

import jax
import jax.numpy as jnp
from jax.experimental import pallas as pl
from jax.experimental.pallas import tpu as pltpu

E = 8
D_MODEL = 1024
D_EXPERT = 1024


def _router_kernel(x_ref, gamma_ref, beta_ref, wg_ref,
                   xn_ref, probs_ref, cw_ref, loss_ref):
    x = x_ref[...]
    mean = jnp.mean(x, axis=-1, keepdims=True)
    var = jnp.mean((x - mean) ** 2, axis=-1, keepdims=True)
    xn = (x - mean) / jnp.sqrt(var + 1e-5) * gamma_ref[...][None, :] + beta_ref[...][None, :]
    xn_ref[...] = xn.astype(jnp.bfloat16)

    logits = jax.lax.dot_general(
        xn, wg_ref[...], (((1,), (1,)), ((), ())),
        preferred_element_type=jnp.float32)
    m = jnp.max(logits, axis=-1, keepdims=True)
    p = jnp.exp(logits - m)
    probs = p / jnp.sum(p, axis=-1, keepdims=True)
    probs_ref[...] = probs

    T = probs.shape[0]
    e_iota = jax.lax.broadcasted_iota(jnp.int32, (T, E), 1)
    m0 = jnp.max(probs, axis=-1, keepdims=True)
    i0 = jnp.min(jnp.where(probs == m0, e_iota, E), axis=-1, keepdims=True)
    oh0 = (e_iota == i0)
    pm = jnp.where(oh0, -1.0, probs)
    m1 = jnp.max(pm, axis=-1, keepdims=True)
    i1 = jnp.min(jnp.where(pm == m1, e_iota, E), axis=-1, keepdims=True)
    oh1 = (e_iota == i1)
    denom = m0 + m1 + 1e-8
    cw = (oh0 * (m0 / denom) + oh1 * (m1 / denom)).astype(jnp.float32)
    cw_ref[...] = cw

    counts = jnp.sum(oh0.astype(jnp.float32) + oh1.astype(jnp.float32), axis=0)
    psum = jnp.sum(probs, axis=0)
    loss = 0.01 * E * jnp.sum(counts * psum) / (T * T)
    loss_ref[...] = jnp.full((1, 128), loss, jnp.float32)


def _ffn_kernel(x_ref, xn_ref, cw_ref, w1_ref, b1_ref, w2_ref, b2_ref, out_ref):
    e = pl.program_id(0)
    xn = xn_ref[...]
    w1 = w1_ref[0].astype(jnp.bfloat16)
    h = jax.lax.dot_general(
        xn, w1, (((1,), (1,)), ((), ())),
        preferred_element_type=jnp.float32) + b1_ref[0]
    h = 0.5 * h * (1.0 + jax.lax.erf(h * 0.7071067811865476))
    h16 = h.astype(jnp.bfloat16)
    w2 = w2_ref[0].astype(jnp.bfloat16)
    yo = jax.lax.dot_general(
        h16, w2, (((1,), (1,)), ((), ())),
        preferred_element_type=jnp.float32) + b2_ref[0]

    T = xn.shape[0]
    lane = jax.lax.broadcasted_iota(jnp.int32, (T, E), 1)
    coef = jnp.sum(cw_ref[...] * (lane == e).astype(jnp.float32),
                   axis=-1, keepdims=True)

    @pl.when(e == 0)
    def _init():
        out_ref[...] = x_ref[...]

    out_ref[...] += coef * yo


def kernel(x, gamma, beta, Wg, W1, b1, W2, b2):
    Bq, T, D = x.shape
    x_flat = x.reshape(Bq * T, D)
    n_tok = Bq * T

    xn16, probs, cw, loss = pl.pallas_call(
        _router_kernel,
        out_shape=(
            jax.ShapeDtypeStruct((n_tok, D), jnp.bfloat16),
            jax.ShapeDtypeStruct((n_tok, E), jnp.float32),
            jax.ShapeDtypeStruct((n_tok, E), jnp.float32),
            jax.ShapeDtypeStruct((1, 128), jnp.float32),
        ),
    )(x_flat, gamma, beta, Wg)

    out = pl.pallas_call(
        _ffn_kernel,
        grid=(E,),
        in_specs=[
            pl.BlockSpec((n_tok, D), lambda e: (0, 0)),
            pl.BlockSpec((n_tok, D), lambda e: (0, 0)),
            pl.BlockSpec((n_tok, E), lambda e: (0, 0)),
            pl.BlockSpec((1, D_EXPERT, D), lambda e: (e, 0, 0)),
            pl.BlockSpec((1, 1, D_EXPERT), lambda e: (e, 0, 0)),
            pl.BlockSpec((1, D, D_EXPERT), lambda e: (e, 0, 0)),
            pl.BlockSpec((1, 1, D), lambda e: (e, 0, 0)),
        ],
        out_specs=pl.BlockSpec((n_tok, D), lambda e: (0, 0)),
        out_shape=jax.ShapeDtypeStruct((n_tok, D), jnp.float32),
    )(x_flat, xn16, cw, W1, b1.reshape(E, 1, D_EXPERT), W2, b2.reshape(E, 1, D))

    return out.reshape(Bq, T, D), loss[0, 0], probs
